# Initial kernel scaffold; baseline (speedup 1.0000x reference)
import jax
import jax.numpy as jnp
from jax.experimental import pallas as pl


def kernel(x, edge_index, edge_attr, W, bias, fc, bf):
    raise NotImplementedError("write your pallas kernel here")


# speedup vs baseline: 18.2442x; 18.2442x over previous
import functools

import jax
import jax.numpy as jnp
from jax import lax
from jax.experimental import pallas as pl
from jax.experimental.pallas import tpu as pltpu
from jax.experimental.pallas import tpu_sc as plsc

NC = 2
NS = 16
CH = 128


def _matmul_call(x, W):
    n, d_in = x.shape
    d_out = W.shape[1]
    dh = d_out // NC
    rm = 400
    grid = (n // rm,)

    def mm(x_ref, w_ref, o_ref):
        hb = jnp.dot(x_ref[...], w_ref[...],
                     preferred_element_type=jnp.float32)
        for c in range(NC):
            o_ref[c] = hb[:, c * dh:(c + 1) * dh]

    return pl.pallas_call(
        mm,
        grid=grid,
        in_specs=[
            pl.BlockSpec((rm, d_in), lambda i: (i, 0)),
            pl.BlockSpec((d_in, d_out), lambda i: (0, 0)),
        ],
        out_specs=pl.BlockSpec((NC, rm, dh), lambda i: (0, i, 0)),
        out_shape=jax.ShapeDtypeStruct((NC, n, dh), jnp.float32),
    )(x, W)


def _sc_scatter_call(h2, colr, rowr, np_rows):
    dh = h2.shape[2]
    nch = colr.shape[1]
    rows_per_tile = np_rows // NS
    wb_chunks = rows_per_tile // CH
    mesh = plsc.VectorSubcoreMesh(core_axis_name="c", subcore_axis_name="s")

    @functools.partial(
        pl.kernel,
        out_type=[
            jax.ShapeDtypeStruct((NC, np_rows, dh), jnp.float32),
            jax.ShapeDtypeStruct((NC, np_rows, 16), jnp.float32),
        ],
        mesh=mesh,
        compiler_params=pltpu.CompilerParams(use_tc_tiling_on_sc=False),
        scratch_types=[
            pltpu.VMEM((nch, CH), jnp.int32),
            pltpu.VMEM((nch, CH), jnp.int32),
            pltpu.VMEM((CH, dh), jnp.float32),
            pltpu.VMEM((CH, dh), jnp.float32),
            pltpu.VMEM((CH, 16), jnp.float32),
            pltpu.VMEM_SHARED((np_rows, dh), jnp.float32),
            pltpu.VMEM_SHARED((np_rows, 16), jnp.float32),
            pltpu.SemaphoreType.DMA,
            pltpu.SemaphoreType.DMA,
        ],
    )
    def sc_body(h_hbm, col_hbm, row_hbm, acc_out, deg_out,
                colv, rowv, buf0, buf1, ones16, acc_sh, deg_sh, sem0, sem1):
        cid = lax.axis_index("c")
        sid = lax.axis_index("s")
        base = sid * rows_per_tile
        table = h_hbm.at[cid]

        pltpu.sync_copy(col_hbm.at[sid], colv)
        pltpu.sync_copy(row_hbm.at[sid], rowv)

        def zrow(i, _):
            for k in range(dh // 16):
                buf0[i, pl.ds(k * 16, 16)] = jnp.zeros((16,), jnp.float32)
            ones16[i, :] = jnp.zeros((16,), jnp.float32)
            return 0

        lax.fori_loop(0, CH, zrow, 0)
        for t in range(wb_chunks):
            sl = pl.ds(base + t * CH, CH)
            pltpu.sync_copy(buf0, acc_sh.at[sl])
            pltpu.sync_copy(ones16, deg_sh.at[sl])

        def orow(i, _):
            ones16[i, :] = jnp.ones((16,), jnp.float32)
            return 0

        lax.fori_loop(0, CH, orow, 0)
        plsc.subcore_barrier()

        pltpu.async_copy(table.at[colv.at[0]], buf0, sem0)
        pltpu.async_copy(table.at[colv.at[1]], buf1, sem1)

        def wait_gather(buf, sem):
            pltpu.make_async_copy(table.at[pl.ds(0, CH)], buf, sem).wait()

        def step(g, _):
            for parity, (buf, sem) in enumerate(((buf0, sem0), (buf1, sem1))):
                j = 2 * g + parity
                wait_gather(buf, sem)
                pltpu.sync_copy(buf, acc_sh.at[rowv.at[j]], add=True)

                @pl.when(cid == parity)
                def _():
                    pltpu.sync_copy(ones16, deg_sh.at[rowv.at[j]], add=True)

                @pl.when(g < nch // 2 - 1)
                def _():
                    pltpu.async_copy(table.at[colv.at[j + 2]], buf, sem)

            return 0

        lax.fori_loop(0, nch // 2, step, 0)
        plsc.subcore_barrier()

        for t in range(wb_chunks):
            sl = pl.ds(base + t * CH, CH)
            pltpu.sync_copy(acc_sh.at[sl], acc_out.at[cid].at[sl])
            pltpu.sync_copy(deg_sh.at[sl], deg_out.at[cid].at[sl])

    return sc_body(h2, colr, rowr)


def _epilogue_call(acc, deg, bias2, fc, bf2, n):
    dh = acc.shape[2]
    d = NC * dh
    rm = 400
    grid = (n // rm,)

    def ep(acc_ref, deg_ref, b_ref, fc_ref, bf_ref, o_ref):
        aa = acc_ref[...]
        dd = deg_ref[...]
        a = jnp.concatenate([aa[0], aa[1]], axis=1)
        dcol = dd[0, :, 0:1] + dd[1, :, 0:1]
        inv = jnp.where(dcol > 0, 1.0 / jnp.where(dcol > 0, dcol, 1.0), 0.0)
        vh = a * inv
        vh = jnp.where(jnp.isnan(vh), jnp.zeros_like(vh), vh)
        vh = vh + b_ref[...]
        s = jax.nn.sigmoid(
            jnp.dot(vh, fc_ref[...], preferred_element_type=jnp.float32)
            + bf_ref[...])
        o_ref[...] = (jnp.where(vh < 0, jnp.zeros_like(vh), vh)
                      + s * jnp.where(vh > 0, jnp.zeros_like(vh), vh))

    return pl.pallas_call(
        ep,
        grid=grid,
        in_specs=[
            pl.BlockSpec((NC, rm, dh), lambda i: (0, i, 0)),
            pl.BlockSpec((NC, rm, 16), lambda i: (0, i, 0)),
            pl.BlockSpec((1, d), lambda i: (0, 0)),
            pl.BlockSpec((d, 1), lambda i: (0, 0)),
            pl.BlockSpec((1, 1), lambda i: (0, 0)),
        ],
        out_specs=pl.BlockSpec((rm, d), lambda i: (i, 0)),
        out_shape=jax.ShapeDtypeStruct((n, d), jnp.float32),
    )(acc, deg, bias2, fc, bf2)


def kernel(x, edge_index, edge_attr, W, bias, fc, bf):
    n = x.shape[0]
    e = edge_index.shape[1]
    np_rows = ((n + NS * CH - 1) // (NS * CH)) * (NS * CH)
    e_pad = ((e + NS * CH - 1) // (NS * CH)) * (NS * CH)
    if (e_pad // (NS * CH)) % 2:
        e_pad += NS * CH
    nch = e_pad // (NS * CH)

    ei = edge_index.astype(jnp.int32)
    row = ei[0]
    col = ei[1]
    padn = e_pad - e
    ar = jnp.arange(padn, dtype=jnp.int32)
    row_p = jnp.concatenate([row, n + (ar % (np_rows - n))])
    col_p = jnp.concatenate([col, ar % n])
    rowr = row_p.reshape(NS, nch, CH)
    colr = col_p.reshape(NS, nch, CH)

    h2 = _matmul_call(x, W)
    acc, deg = _sc_scatter_call(h2, colr, rowr, np_rows)
    out = _epilogue_call(acc, deg, bias.reshape(1, -1), fc,
                         bf.reshape(1, 1), n)
    return out


# speedup vs baseline: 21.7426x; 1.1918x over previous
import functools

import numpy as np_host

import jax
import jax.numpy as jnp
from jax import lax
from jax.experimental import pallas as pl
from jax.experimental.pallas import tpu as pltpu
from jax.experimental.pallas import tpu_sc as plsc

NC = 2
NS = 16
CH = 128


def _matmul_call(x, W):
    n, d_in = x.shape
    d_out = W.shape[1]
    dh = d_out // NC
    rm = 2000
    grid = (n // rm,)

    def mm(x_ref, w_ref, o_ref):
        hb = jnp.dot(x_ref[...], w_ref[...],
                     preferred_element_type=jnp.float32)
        for c in range(NC):
            o_ref[c] = hb[:, c * dh:(c + 1) * dh]

    return pl.pallas_call(
        mm,
        grid=grid,
        in_specs=[
            pl.BlockSpec((rm, d_in), lambda i: (i, 0)),
            pl.BlockSpec((d_in, d_out), lambda i: (0, 0)),
        ],
        out_specs=pl.BlockSpec((NC, rm, dh), lambda i: (0, i, 0)),
        out_shape=jax.ShapeDtypeStruct((NC, n, dh), jnp.float32),
    )(x, W)


def _sc_scatter_call(h2, colr, rowr, np_rows):
    dh = h2.shape[2]
    nch = colr.shape[1]
    rows_per_tile = np_rows // NS
    wb_chunks = rows_per_tile // CH
    mesh = plsc.VectorSubcoreMesh(core_axis_name="c", subcore_axis_name="s")

    @functools.partial(
        pl.kernel,
        out_type=[
            jax.ShapeDtypeStruct((NC, np_rows, dh), jnp.float32),
            jax.ShapeDtypeStruct((NC, np_rows, 16), jnp.float32),
        ],
        mesh=mesh,
        compiler_params=pltpu.CompilerParams(use_tc_tiling_on_sc=False),
        scratch_types=[
            pltpu.VMEM((nch, CH), jnp.int32),
            pltpu.VMEM((nch, CH), jnp.int32),
            pltpu.VMEM((CH, dh), jnp.float32),
            pltpu.VMEM((CH, dh), jnp.float32),
            pltpu.VMEM((CH, dh), jnp.float32),
            pltpu.VMEM((CH, 16), jnp.float32),
            pltpu.VMEM_SHARED((np_rows, dh), jnp.float32),
            pltpu.VMEM_SHARED((np_rows, 16), jnp.float32),
            [pltpu.SemaphoreType.DMA] * 3,
            [pltpu.SemaphoreType.DMA] * 3,
        ],
    )
    def sc_body(h_hbm, col_hbm, row_hbm, acc_out, deg_out,
                colv, rowv, buf0, buf1, buf2, ones16, acc_sh, deg_sh,
                gsem, ssem):
        cid = lax.axis_index("c")
        sid = lax.axis_index("s")
        base = sid * rows_per_tile
        table = h_hbm.at[cid]
        bufs = (buf0, buf1, buf2)

        pltpu.sync_copy(col_hbm.at[sid], colv)
        pltpu.sync_copy(row_hbm.at[sid], rowv)
        pltpu.async_copy(table.at[colv.at[0]], buf0, gsem[0])
        pltpu.async_copy(table.at[colv.at[1]], buf1, gsem[1])

        def zrow(i, _):
            for k in range(dh // 16):
                buf2[i, pl.ds(k * 16, 16)] = jnp.zeros((16,), jnp.float32)
            ones16[i, :] = jnp.zeros((16,), jnp.float32)
            return 0

        lax.fori_loop(0, CH, zrow, 0)
        for t in range(wb_chunks):
            sl = pl.ds(base + t * CH, CH)
            pltpu.sync_copy(buf2, acc_sh.at[sl])
            pltpu.sync_copy(ones16, deg_sh.at[sl])

        def orow(i, _):
            ones16[i, :] = jnp.ones((16,), jnp.float32)
            return 0

        lax.fori_loop(0, CH, orow, 0)
        plsc.subcore_barrier()

        def wait_gather(b):
            pltpu.make_async_copy(table.at[pl.ds(0, CH)], bufs[b], gsem[b])\
                .wait()

        def wait_scatter(b, j):
            pltpu.make_async_copy(bufs[b], acc_sh.at[rowv.at[j]], ssem[b])\
                .wait()

        def step(g, _):
            for b in range(3):
                j = 3 * g + b
                wait_gather(b)
                pltpu.async_copy(bufs[b], acc_sh.at[rowv.at[j]], ssem[b],
                                 add=True)

                @pl.when(cid == j % 2)
                def _():
                    pltpu.sync_copy(ones16, deg_sh.at[rowv.at[j]], add=True)

                prev = (b - 1) % 3
                nxt = (b + 2) % 3

                if b == 0:
                    @pl.when(g >= 1)
                    def _():
                        wait_scatter(prev, j - 1)
                    pltpu.async_copy(table.at[colv.at[j + 2]], bufs[nxt],
                                     gsem[nxt])
                else:
                    @pl.when(g < nch // 3 - 1)
                    def _():
                        wait_scatter(prev, j - 1)
                        pltpu.async_copy(table.at[colv.at[j + 2]],
                                         bufs[nxt], gsem[nxt])

            return 0

        lax.fori_loop(0, nch // 3, step, 0)
        for j in (nch - 3, nch - 2, nch - 1):
            wait_scatter(j % 3, j)
        plsc.subcore_barrier()

        for t in range(wb_chunks):
            sl = pl.ds(base + t * CH, CH)
            pltpu.sync_copy(acc_sh.at[sl], acc_out.at[cid].at[sl])
            pltpu.sync_copy(deg_sh.at[sl], deg_out.at[cid].at[sl])

    return sc_body(h2, colr, rowr)


def _epilogue_call(acc, deg, bias2, fc, bf2, n):
    dh = acc.shape[2]
    d = NC * dh
    rm = 2000
    grid = (n // rm,)

    def ep(acc_ref, deg_ref, b_ref, fc_ref, bf_ref, o_ref):
        aa = acc_ref[...]
        dd = deg_ref[...]
        a = jnp.concatenate([aa[0], aa[1]], axis=1)
        dcol = dd[0, :, 0:1] + dd[1, :, 0:1]
        inv = jnp.where(dcol > 0, 1.0 / jnp.where(dcol > 0, dcol, 1.0), 0.0)
        vh = a * inv
        vh = jnp.where(jnp.isnan(vh), jnp.zeros_like(vh), vh)
        vh = vh + b_ref[...]
        s = jax.nn.sigmoid(
            jnp.dot(vh, fc_ref[...], preferred_element_type=jnp.float32)
            + bf_ref[...])
        o_ref[...] = (jnp.where(vh < 0, jnp.zeros_like(vh), vh)
                      + s * jnp.where(vh > 0, jnp.zeros_like(vh), vh))

    return pl.pallas_call(
        ep,
        grid=grid,
        in_specs=[
            pl.BlockSpec((NC, rm, dh), lambda i: (0, i, 0)),
            pl.BlockSpec((NC, rm, 16), lambda i: (0, i, 0)),
            pl.BlockSpec((1, d), lambda i: (0, 0)),
            pl.BlockSpec((d, 1), lambda i: (0, 0)),
            pl.BlockSpec((1, 1), lambda i: (0, 0)),
        ],
        out_specs=pl.BlockSpec((rm, d), lambda i: (i, 0)),
        out_shape=jax.ShapeDtypeStruct((n, d), jnp.float32),
    )(acc, deg, bias2, fc, bf2)


def kernel(x, edge_index, edge_attr, W, bias, fc, bf):
    n = x.shape[0]
    e = edge_index.shape[1]
    np_rows = ((n + NS * CH - 1) // (NS * CH)) * (NS * CH)
    blk = NS * CH * 3
    e_pad = ((e + blk - 1) // blk) * blk
    nch = e_pad // (NS * CH)

    ei = edge_index.astype(jnp.int32)
    row = ei[0]
    col = ei[1]
    padn = e_pad - e
    ar = np_host.arange(padn, dtype=np_host.int32)
    row_pad = jnp.asarray(n + (ar % (np_rows - n)), dtype=jnp.int32)
    col_pad = jnp.asarray(ar % n, dtype=jnp.int32)
    rowr = jnp.concatenate([row, row_pad]).reshape(NS, nch, CH)
    colr = jnp.concatenate([col, col_pad]).reshape(NS, nch, CH)

    h2 = _matmul_call(x, W)
    acc, deg = _sc_scatter_call(h2, colr, rowr, np_rows)
    out = _epilogue_call(acc, deg, bias.reshape(1, -1), fc,
                         bf.reshape(1, 1), n)
    return out
